# baseline (speedup 1.0000x reference)
import functools

import jax
import jax.numpy as jnp
from jax import lax
from jax.experimental import pallas as pl
from jax.experimental.pallas import tpu as pltpu
from jax.experimental.pallas import tpu_sc as plsc

BATCH, SEQ = 4096, 20
NTOK = BATCH * SEQ
V = 1000
D = 4
DP = 16
PK = 128
PR = NTOK * DP // PK

NC, NS = 2, 16
NW = NC * NS
TPW = NTOK // NW
IDXC = 128
NCH = TPW // IDXC


@functools.partial(
    pl.kernel,
    out_type=jax.ShapeDtypeStruct((NTOK, DP), jnp.float32),
    mesh=plsc.VectorSubcoreMesh(core_axis_name="c", subcore_axis_name="s"),
    compiler_params=pltpu.CompilerParams(use_tc_tiling_on_sc=False),
    scratch_types=[
        pltpu.VMEM((TPW,), jnp.int32),
        pltpu.VMEM((TPW, DP), jnp.float32),
        pltpu.SemaphoreType.DMA,
    ],
)
def _sc_gather(idx_hbm, embed_hbm, emb_hbm, idx_v, rows_v, sem):
    wid = lax.axis_index("s") * NC + lax.axis_index("c")
    base = wid * TPW
    pltpu.sync_copy(idx_hbm.at[pl.ds(base, TPW)], idx_v)
    descs = [
        pltpu.async_copy(
            embed_hbm.at[idx_v.at[pl.ds(c * IDXC, IDXC)]],
            rows_v.at[pl.ds(c * IDXC, IDXC)], sem)
        for c in range(NCH)
    ]
    for d in descs:
        d.wait()
    pltpu.sync_copy(rows_v, emb_hbm.at[pl.ds(base, TPW)])


BB = BATCH
NCK = 5
CR = V // NCK


def _out_dmas(acc, out_hbm, t, sem):
    slot = lax.rem(t, 2)
    return [
        pltpu.make_async_copy(
            acc.at[slot, pl.ds(c * CR, CR)],
            out_hbm.at[t, pl.ds(c * CR, CR)],
            sem.at[slot, c])
        for c in range(NCK)
    ]


def _mm_body(emb_ref, w_ref, b_ref, out_hbm, acc, sem):
    s = pl.program_id(0)

    @pl.when(s >= 2)
    def _():
        for d in _out_dmas(acc, out_hbm, s - 2, sem):
            d.wait()

    res = lax.dot_general(
        w_ref[...], emb_ref[...],
        dimension_numbers=(((1,), (1,)), ((), ())),
        preferred_element_type=jnp.float32) + b_ref[...]

    @pl.when(lax.rem(s, 2) == 0)
    def _():
        acc[0] = res

    @pl.when(lax.rem(s, 2) == 1)
    def _():
        acc[1] = res

    for d in _out_dmas(acc, out_hbm, s, sem):
        d.start()

    @pl.when(s == SEQ - 1)
    def _():
        for t in (SEQ - 2, SEQ - 1):
            for d in _out_dmas(acc, out_hbm, t, sem):
                d.wait()


def kernel(x, embed, W, b):
    embed16 = jnp.pad(embed, ((0, 0), (0, DP - D)))
    w16 = jnp.pad(W, ((0, 0), (0, DP - D)))
    idx = x.astype(jnp.int32).T.reshape(NTOK)
    emb = _sc_gather(idx, embed16)
    out_t = pl.pallas_call(
        _mm_body,
        grid=(SEQ,),
        in_specs=[
            pl.BlockSpec((BB, DP), lambda s: (s, 0)),
            pl.BlockSpec((V, DP), lambda s: (0, 0)),
            pl.BlockSpec((V, 1), lambda s: (0, 0)),
        ],
        out_specs=pl.BlockSpec(memory_space=pl.ANY),
        out_shape=jax.ShapeDtypeStruct((SEQ, V, BATCH), jnp.float32),
        scratch_shapes=[
            pltpu.VMEM((2, V, BB), jnp.float32),
            pltpu.SemaphoreType.DMA((2, NCK)),
        ],
    )(emb, w16, b.reshape(V, 1))
    return out_t.transpose(2, 0, 1)


# speedup vs baseline: 1.0089x; 1.0089x over previous
import functools

import jax
import jax.numpy as jnp
from jax import lax
from jax.experimental import pallas as pl
from jax.experimental.pallas import tpu as pltpu
from jax.experimental.pallas import tpu_sc as plsc

BATCH, SEQ = 4096, 20
NTOK = BATCH * SEQ
V = 1000
D = 4
DP = 16
PK = 128
PR = NTOK * DP // PK

NC, NS = 2, 16
NW = NC * NS
TPW = NTOK // NW
IDXC = 128
NCH = TPW // IDXC


@functools.partial(
    pl.kernel,
    out_type=jax.ShapeDtypeStruct((NTOK, DP), jnp.float32),
    mesh=plsc.VectorSubcoreMesh(core_axis_name="c", subcore_axis_name="s"),
    compiler_params=pltpu.CompilerParams(use_tc_tiling_on_sc=False),
    scratch_types=[
        pltpu.VMEM((TPW,), jnp.int32),
        pltpu.VMEM((TPW, DP), jnp.float32),
        pltpu.SemaphoreType.DMA,
    ],
)
def _sc_gather(idx_hbm, embed_hbm, emb_hbm, idx_v, rows_v, sem):
    wid = lax.axis_index("s") * NC + lax.axis_index("c")
    base = wid * TPW
    pltpu.sync_copy(idx_hbm.at[pl.ds(base, TPW)], idx_v)
    descs = [
        pltpu.async_copy(
            embed_hbm.at[idx_v.at[pl.ds(c * IDXC, IDXC)]],
            rows_v.at[pl.ds(c * IDXC, IDXC)], sem)
        for c in range(NCH)
    ]
    for d in descs:
        d.wait()
    pltpu.sync_copy(rows_v, emb_hbm.at[pl.ds(base, TPW)])


BB = 4096
NI = BATCH // BB


def _mm_body(emb_ref, w_ref, b_ref, out_ref):
    out_ref[...] = lax.dot_general(
        w_ref[...], emb_ref[...],
        dimension_numbers=(((1,), (1,)), ((), ())),
        preferred_element_type=jnp.float32) + b_ref[...]


def kernel(x, embed, W, b):
    embed16 = jnp.pad(embed, ((0, 0), (0, DP - D)))
    w16 = jnp.pad(W, ((0, 0), (0, DP - D)))
    idx = x.astype(jnp.int32).T.reshape(NTOK)
    emb = _sc_gather(idx, embed16)
    out_t = pl.pallas_call(
        _mm_body,
        grid=(SEQ, NI),
        in_specs=[
            pl.BlockSpec((BB, DP), lambda s, i: (s * NI + i, 0)),
            pl.BlockSpec((V, DP), lambda s, i: (0, 0)),
            pl.BlockSpec((V, 1), lambda s, i: (0, 0)),
        ],
        out_specs=pl.BlockSpec((None, V, BB), lambda s, i: (s, 0, i)),
        out_shape=jax.ShapeDtypeStruct((SEQ, V, BATCH), jnp.float32),
    )(emb, w16, b.reshape(V, 1))
    return out_t.transpose(2, 0, 1)
